# baseline (speedup 1.0000x reference)
import functools

import jax
import jax.numpy as jnp
from jax import lax
from jax.experimental import pallas as pl
from jax.experimental.pallas import tpu as pltpu
from jax.experimental.pallas import tpu_sc as plsc

PAD_IDX = 0
BATCH = 4096
SEQ = 50
EMBED_DIM = 128
N = BATCH * SEQ

NUM_CORES = 2
NUM_SUBCORES = 16
NW = NUM_CORES * NUM_SUBCORES
PER_W = N // NW
CHUNK = 128
NCHUNK = PER_W // CHUNK


NBUF = 5
NGROUP = NCHUNK // NBUF
LOOKAHEAD = 3


@functools.partial(
    pl.kernel,
    mesh=plsc.VectorSubcoreMesh(core_axis_name="c", subcore_axis_name="s"),
    out_type=jax.ShapeDtypeStruct((N, EMBED_DIM), jnp.float32),
    scratch_types=[
        pltpu.VMEM((PER_W,), jnp.int32),
    ]
    + [pltpu.VMEM((CHUNK, EMBED_DIM), jnp.float32)] * NBUF
    + [pltpu.SemaphoreType.DMA] * (2 * NBUF),
)
def _sc_gather(idx_hbm, table_hbm, out_hbm, idx_v, *bufs_and_sems):
    rows = bufs_and_sems[:NBUF]
    gsem = bufs_and_sems[NBUF:2 * NBUF]
    wsem = bufs_and_sems[2 * NBUF:]
    wid = lax.axis_index("s") * NUM_CORES + lax.axis_index("c")
    base = wid * PER_W
    pltpu.sync_copy(idx_hbm.at[pl.ds(base, PER_W)], idx_v)

    def gather(c, b):
        pltpu.async_copy(
            table_hbm.at[idx_v.at[pl.ds(c * CHUNK, CHUNK)]], rows[b], gsem[b])

    def write(c, b):
        pltpu.async_copy(rows[b], out_hbm.at[pl.ds(base + c * CHUNK, CHUNK)],
                         wsem[b])

    def gather_wait(b):
        pltpu.make_async_copy(
            table_hbm.at[idx_v.at[pl.ds(0, CHUNK)]], rows[b], gsem[b]).wait()

    def write_wait(b):
        pltpu.make_async_copy(
            rows[b], out_hbm.at[pl.ds(0, CHUNK)], wsem[b]).wait()

    for b in range(LOOKAHEAD):
        gather(b, b)

    def group(g, carry):
        for b in range(NBUF):
            c = g * NBUF + b
            gather_wait(b)
            write(c, b)
            nb = (b + LOOKAHEAD) % NBUF
            if b < NBUF - LOOKAHEAD:
                @pl.when(g > 0)
                def _():
                    write_wait(nb)

                gather(c + LOOKAHEAD, nb)
            else:
                @pl.when(g < NGROUP - 1)
                def _():
                    write_wait(nb)
                    gather(c + LOOKAHEAD, nb)
        return carry

    lax.fori_loop(0, NGROUP, group, 0)
    for b in range(NBUF):
        write_wait(b)


def kernel(tokens, table):
    idx = tokens.astype(jnp.int32).T.reshape(N)
    out = _sc_gather(idx, table)
    emb = out.reshape(SEQ, BATCH, EMBED_DIM)
    mask = (tokens != PAD_IDX).T
    return emb, mask


# speedup vs baseline: 1.0081x; 1.0081x over previous
import functools

import jax
import jax.numpy as jnp
from jax import lax
from jax.experimental import pallas as pl
from jax.experimental.pallas import tpu as pltpu
from jax.experimental.pallas import tpu_sc as plsc

PAD_IDX = 0
BATCH = 4096
SEQ = 50
EMBED_DIM = 128
N = BATCH * SEQ

NUM_CORES = 2
NUM_SUBCORES = 16
NW = NUM_CORES * NUM_SUBCORES
PER_W = N // NW
CHUNK = 128
NCHUNK = PER_W // CHUNK


NBUF = 5
NGROUP = NCHUNK // NBUF
LOOKAHEAD = 2


@functools.partial(
    pl.kernel,
    mesh=plsc.VectorSubcoreMesh(core_axis_name="c", subcore_axis_name="s"),
    out_type=jax.ShapeDtypeStruct((N, EMBED_DIM), jnp.float32),
    scratch_types=[
        pltpu.VMEM((PER_W,), jnp.int32),
    ]
    + [pltpu.VMEM((CHUNK, EMBED_DIM), jnp.float32)] * NBUF
    + [pltpu.SemaphoreType.DMA] * (2 * NBUF),
)
def _sc_gather(idx_hbm, table_hbm, out_hbm, idx_v, *bufs_and_sems):
    rows = bufs_and_sems[:NBUF]
    gsem = bufs_and_sems[NBUF:2 * NBUF]
    wsem = bufs_and_sems[2 * NBUF:]
    wid = lax.axis_index("s") * NUM_CORES + lax.axis_index("c")
    base = wid * PER_W
    pltpu.sync_copy(idx_hbm.at[pl.ds(base, PER_W)], idx_v)

    def gather(c, b):
        pltpu.async_copy(
            table_hbm.at[idx_v.at[pl.ds(c * CHUNK, CHUNK)]], rows[b], gsem[b])

    def write(c, b):
        pltpu.async_copy(rows[b], out_hbm.at[pl.ds(base + c * CHUNK, CHUNK)],
                         wsem[b])

    def gather_wait(b):
        pltpu.make_async_copy(
            table_hbm.at[idx_v.at[pl.ds(0, CHUNK)]], rows[b], gsem[b]).wait()

    def write_wait(b):
        pltpu.make_async_copy(
            rows[b], out_hbm.at[pl.ds(0, CHUNK)], wsem[b]).wait()

    for b in range(LOOKAHEAD):
        gather(b, b)

    def group(g, carry):
        for b in range(NBUF):
            c = g * NBUF + b
            gather_wait(b)
            write(c, b)
            nb = (b + LOOKAHEAD) % NBUF
            if b < NBUF - LOOKAHEAD:
                @pl.when(g > 0)
                def _():
                    write_wait(nb)

                gather(c + LOOKAHEAD, nb)
            else:
                @pl.when(g < NGROUP - 1)
                def _():
                    write_wait(nb)
                    gather(c + LOOKAHEAD, nb)
        return carry

    lax.fori_loop(0, NGROUP, group, 0)
    for b in range(NBUF):
        write_wait(b)


def kernel(tokens, table):
    idx = tokens.astype(jnp.int32).T.reshape(N)
    out = _sc_gather(idx, table)
    emb = out.reshape(SEQ, BATCH, EMBED_DIM)
    mask = (tokens != PAD_IDX).T
    return emb, mask
